# baseline (speedup 1.0000x reference)
import functools

import jax
import jax.numpy as jnp
from jax import lax
from jax.experimental import pallas as pl
from jax.experimental.pallas import tpu as pltpu
from jax.experimental.pallas import tpu_sc as plsc

NC = 2
NS = 16
CHUNK = 128
R0, R1 = 5, 1
NSLICE = NS * (R0 + R1)


def _round_up(x, m):
    return (x + m - 1) // m * m


@functools.lru_cache(maxsize=None)
def _make_sc_aggregate(n, d, npad, cps):
    rps = npad // NS
    n_pair = (cps + 1) // 2

    mesh = plsc.VectorSubcoreMesh(core_axis_name="c", subcore_axis_name="s")

    @functools.partial(
        pl.kernel,
        mesh=mesh,
        out_type=jax.ShapeDtypeStruct((NC, npad, d), jnp.float32),
        scratch_types=[
            pltpu.VMEM((cps, CHUNK), jnp.int32),
            pltpu.VMEM((cps, CHUNK), jnp.int32),
            pltpu.VMEM((CHUNK, d), jnp.float32),
            pltpu.VMEM((CHUNK, d), jnp.float32),
            pltpu.VMEM_SHARED((npad, d), jnp.float32),
            pltpu.SemaphoreType.DMA,
            pltpu.SemaphoreType.DMA,
        ],
    )
    def sc_agg(src_hbm, dst_hbm, feat_hbm, zeros_hbm, out_hbm,
               src_v, dst_v, rows0, rows1, agg_sh, sem0, sem1):
        cid = lax.axis_index("c")
        sid = lax.axis_index("s")
        row0 = sid * rps

        pltpu.sync_copy(zeros_hbm, agg_sh.at[pl.ds(row0, rps)])
        plsc.subcore_barrier()

        def do_slice(slice_id):
            pltpu.sync_copy(src_hbm.at[slice_id], src_v)
            pltpu.sync_copy(dst_hbm.at[slice_id], dst_v)
            pltpu.async_copy(feat_hbm.at[src_v.at[0]], rows0, sem0)

            def pair_body(j, carry):
                c0 = 2 * j
                c1 = jnp.minimum(2 * j + 1, cps - 1)
                c2 = jnp.minimum(2 * j + 2, cps - 1)
                pltpu.async_copy(feat_hbm.at[src_v.at[c1]], rows1, sem1)
                pltpu.make_async_copy(
                    feat_hbm.at[src_v.at[c0]], rows0, sem0).wait()
                pltpu.sync_copy(rows0, agg_sh.at[dst_v.at[c0]], add=True)
                pltpu.async_copy(feat_hbm.at[src_v.at[c2]], rows0, sem0)
                pltpu.make_async_copy(
                    feat_hbm.at[src_v.at[c1]], rows1, sem1).wait()

                @pl.when(2 * j + 1 <= cps - 1)
                def _():
                    pltpu.sync_copy(rows1, agg_sh.at[dst_v.at[c1]], add=True)
                return carry
            lax.fori_loop(0, n_pair, pair_body, 0)
            pltpu.make_async_copy(feat_hbm.at[src_v.at[0]], rows0, sem0).wait()

        @pl.when(cid == 0)
        def _():
            for q in range(R0):
                do_slice(sid * R0 + q)

        @pl.when(cid == 1)
        def _():
            for q in range(R1):
                do_slice(NS * R0 + sid * R1 + q)

        plsc.subcore_barrier()

        pltpu.sync_copy(agg_sh.at[pl.ds(row0, rps)],
                        out_hbm.at[cid, pl.ds(row0, rps)])

    return sc_agg


def _make_mlp(n, d, npad):
    def _mlp_body(feat, aggs, w1, b1, g, be, w2, b2, out):
        h = feat[...] + aggs[0, :n, :] + aggs[1, :n, :]
        h = lax.dot_general(h, w1[...], (((1,), (1,)), ((), ())),
                            preferred_element_type=jnp.float32) + b1[...]
        mean = jnp.mean(h, axis=0, keepdims=True)
        c = h - mean
        var = jnp.mean(c * c, axis=0, keepdims=True)
        h = c * lax.rsqrt(var + 1e-5) * g[...] + be[...]
        h = jnp.maximum(h, 0.0)
        out[...] = lax.dot_general(h, w2[...], (((1,), (1,)), ((), ())),
                                   preferred_element_type=jnp.float32) + b2[...]

    return pl.pallas_call(
        _mlp_body, out_shape=jax.ShapeDtypeStruct((n, d), jnp.float32))


def kernel(nfeat, edge_index, W1, b1, bn_gamma, bn_beta, W2, b2):
    n, d = nfeat.shape
    e = edge_index.shape[1]
    npad = _round_up(n + 1, NS * 8)
    cps = _round_up(e, NSLICE * CHUNK) // (NSLICE * CHUNK)
    epad = NSLICE * cps * CHUNK
    src = edge_index[0]
    dst = edge_index[1]
    if epad > e:
        src = jnp.concatenate([src, jnp.zeros((epad - e,), jnp.int32)])
        dst = jnp.concatenate([dst, jnp.full((epad - e,), n, jnp.int32)])
    src = src.reshape(NSLICE, cps, CHUNK)
    dst = dst.reshape(NSLICE, cps, CHUNK)
    zeros = jnp.zeros((npad // NS, d), jnp.float32)
    aggs = _make_sc_aggregate(n, d, npad, cps)(src, dst, nfeat, zeros)
    return _make_mlp(n, d, npad)(
        nfeat, aggs, W1, b1.reshape(1, d), bn_gamma.reshape(1, d),
        bn_beta.reshape(1, d), W2, b2.reshape(1, d))


# speedup vs baseline: 4.1269x; 4.1269x over previous
import functools

import jax
import jax.numpy as jnp
from jax import lax
from jax.experimental import pallas as pl
from jax.experimental.pallas import tpu as pltpu
from jax.experimental.pallas import tpu_sc as plsc

NC = 2
NS = 16
CHUNK = 128
R0, R1 = 2, 2
NSLICE = NS * (R0 + R1)


def _round_up(x, m):
    return (x + m - 1) // m * m


@functools.lru_cache(maxsize=None)
def _make_sc_aggregate(n, d, npad, cps):
    rps = npad // NS
    n_pair = (cps + 1) // 2

    mesh = plsc.VectorSubcoreMesh(core_axis_name="c", subcore_axis_name="s")

    @functools.partial(
        pl.kernel,
        mesh=mesh,
        out_type=jax.ShapeDtypeStruct((NC, npad, d), jnp.float32),
        scratch_types=[
            pltpu.VMEM((cps, CHUNK), jnp.int32),
            pltpu.VMEM((cps, CHUNK), jnp.int32),
            pltpu.VMEM((CHUNK, d), jnp.float32),
            pltpu.VMEM((CHUNK, d), jnp.float32),
            pltpu.VMEM_SHARED((npad, d), jnp.float32),
            pltpu.SemaphoreType.DMA,
            pltpu.SemaphoreType.DMA,
        ],
    )
    def sc_agg(src_hbm, dst_hbm, feat_hbm, zeros_hbm, out_hbm,
               src_v, dst_v, rows0, rows1, agg_sh, sem0, sem1):
        cid = lax.axis_index("c")
        sid = lax.axis_index("s")
        row0 = sid * rps

        pltpu.sync_copy(zeros_hbm, agg_sh.at[pl.ds(row0, rps)])
        plsc.subcore_barrier()

        def do_slice(slice_id):
            pltpu.sync_copy(src_hbm.at[slice_id], src_v)
            pltpu.sync_copy(dst_hbm.at[slice_id], dst_v)
            pltpu.async_copy(feat_hbm.at[src_v.at[0]], rows0, sem0)

            def pair_body(j, carry):
                c0 = 2 * j
                c1 = jnp.minimum(2 * j + 1, cps - 1)
                c2 = jnp.minimum(2 * j + 2, cps - 1)
                pltpu.async_copy(feat_hbm.at[src_v.at[c1]], rows1, sem1)
                pltpu.make_async_copy(
                    feat_hbm.at[src_v.at[c0]], rows0, sem0).wait()
                pltpu.sync_copy(rows0, agg_sh.at[dst_v.at[c0]], add=True)
                pltpu.async_copy(feat_hbm.at[src_v.at[c2]], rows0, sem0)
                pltpu.make_async_copy(
                    feat_hbm.at[src_v.at[c1]], rows1, sem1).wait()

                @pl.when(2 * j + 1 <= cps - 1)
                def _():
                    pltpu.sync_copy(rows1, agg_sh.at[dst_v.at[c1]], add=True)
                return carry
            lax.fori_loop(0, n_pair, pair_body, 0)
            pltpu.make_async_copy(feat_hbm.at[src_v.at[0]], rows0, sem0).wait()

        @pl.when(cid == 0)
        def _():
            for q in range(R0):
                do_slice(sid * R0 + q)

        @pl.when(cid == 1)
        def _():
            for q in range(R1):
                do_slice(NS * R0 + sid * R1 + q)

        plsc.subcore_barrier()

        pltpu.sync_copy(agg_sh.at[pl.ds(row0, rps)],
                        out_hbm.at[cid, pl.ds(row0, rps)])

    return sc_agg


def _make_mlp(n, d, npad):
    def _mlp_body(feat, aggs, w1, b1, g, be, w2, b2, out):
        h = feat[...] + aggs[0, :n, :] + aggs[1, :n, :]
        h = lax.dot_general(h, w1[...], (((1,), (1,)), ((), ())),
                            preferred_element_type=jnp.float32) + b1[...]
        mean = jnp.mean(h, axis=0, keepdims=True)
        c = h - mean
        var = jnp.mean(c * c, axis=0, keepdims=True)
        h = c * lax.rsqrt(var + 1e-5) * g[...] + be[...]
        h = jnp.maximum(h, 0.0)
        out[...] = lax.dot_general(h, w2[...], (((1,), (1,)), ((), ())),
                                   preferred_element_type=jnp.float32) + b2[...]

    return pl.pallas_call(
        _mlp_body, out_shape=jax.ShapeDtypeStruct((n, d), jnp.float32))


def kernel(nfeat, edge_index, W1, b1, bn_gamma, bn_beta, W2, b2):
    n, d = nfeat.shape
    e = edge_index.shape[1]
    npad = _round_up(n + 1, NS * 8)
    cps = _round_up(e, NSLICE * CHUNK) // (NSLICE * CHUNK)
    epad = NSLICE * cps * CHUNK
    src = edge_index[0]
    dst = edge_index[1]
    if epad > e:
        i = jnp.arange(epad - e, dtype=jnp.int32)
        src = jnp.concatenate([src, i % n])
        dst = jnp.concatenate([dst, n + i % (npad - n)])
    src = src.reshape(NSLICE, cps, CHUNK)
    dst = dst.reshape(NSLICE, cps, CHUNK)
    zeros = jnp.zeros((npad // NS, d), jnp.float32)
    aggs = _make_sc_aggregate(n, d, npad, cps)(src, dst, nfeat, zeros)
    return _make_mlp(n, d, npad)(
        nfeat, aggs, W1, b1.reshape(1, d), bn_gamma.reshape(1, d),
        bn_beta.reshape(1, d), W2, b2.reshape(1, d))


# speedup vs baseline: 4.1367x; 1.0024x over previous
import functools

import jax
import jax.numpy as jnp
from jax import lax
from jax.experimental import pallas as pl
from jax.experimental.pallas import tpu as pltpu
from jax.experimental.pallas import tpu_sc as plsc

NC = 2
NS = 16
CHUNK = 128
R0, R1 = 2, 2
NSLICE = NS * (R0 + R1)


def _round_up(x, m):
    return (x + m - 1) // m * m


@functools.lru_cache(maxsize=None)
def _make_sc_aggregate(n, d, npad, cps):
    rps = npad // NS
    n_pair = (cps + 1) // 2

    mesh = plsc.VectorSubcoreMesh(core_axis_name="c", subcore_axis_name="s")

    @functools.partial(
        pl.kernel,
        mesh=mesh,
        out_type=jax.ShapeDtypeStruct((NC * npad, d), jnp.float32),
        scratch_types=[
            pltpu.VMEM((cps, CHUNK), jnp.int32),
            pltpu.VMEM((cps, CHUNK), jnp.int32),
            pltpu.VMEM((CHUNK, d), jnp.float32),
            pltpu.VMEM((CHUNK, d), jnp.float32),
            pltpu.VMEM_SHARED((npad, d), jnp.float32),
            pltpu.SemaphoreType.DMA,
            pltpu.SemaphoreType.DMA,
        ],
    )
    def sc_agg(src_hbm, dst_hbm, feat_hbm, zeros_hbm, out_hbm,
               src_v, dst_v, rows0, rows1, agg_sh, sem0, sem1):
        cid = lax.axis_index("c")
        sid = lax.axis_index("s")
        row0 = sid * rps

        pltpu.sync_copy(zeros_hbm, agg_sh.at[pl.ds(row0, rps)])
        plsc.subcore_barrier()

        def do_slice(slice_id):
            pltpu.sync_copy(src_hbm.at[slice_id], src_v)
            pltpu.sync_copy(dst_hbm.at[slice_id], dst_v)
            pltpu.async_copy(feat_hbm.at[src_v.at[0]], rows0, sem0)

            def pair_body(j, carry):
                c0 = 2 * j
                c1 = jnp.minimum(2 * j + 1, cps - 1)
                c2 = jnp.minimum(2 * j + 2, cps - 1)
                pltpu.async_copy(feat_hbm.at[src_v.at[c1]], rows1, sem1)
                pltpu.make_async_copy(
                    feat_hbm.at[src_v.at[c0]], rows0, sem0).wait()
                pltpu.sync_copy(rows0, agg_sh.at[dst_v.at[c0]], add=True)
                pltpu.async_copy(feat_hbm.at[src_v.at[c2]], rows0, sem0)
                pltpu.make_async_copy(
                    feat_hbm.at[src_v.at[c1]], rows1, sem1).wait()

                @pl.when(2 * j + 1 <= cps - 1)
                def _():
                    pltpu.sync_copy(rows1, agg_sh.at[dst_v.at[c1]], add=True)
                return carry
            lax.fori_loop(0, n_pair, pair_body, 0)
            pltpu.make_async_copy(feat_hbm.at[src_v.at[0]], rows0, sem0).wait()

        @pl.when(cid == 0)
        def _():
            for q in range(R0):
                do_slice(sid * R0 + q)

        @pl.when(cid == 1)
        def _():
            for q in range(R1):
                do_slice(NS * R0 + sid * R1 + q)

        plsc.subcore_barrier()

        pltpu.sync_copy(agg_sh.at[pl.ds(row0, rps)],
                        out_hbm.at[pl.ds(cid * npad + row0, rps)])

    return sc_agg


def _make_mlp(n, d, npad):
    def _mlp_body(feat, aggs, w1, b1, g, be, w2, b2, out):
        h = feat[...] + aggs[:n, :] + aggs[npad:npad + n, :]
        h = lax.dot_general(h, w1[...], (((1,), (1,)), ((), ())),
                            preferred_element_type=jnp.float32) + b1[...]
        mean = jnp.mean(h, axis=0, keepdims=True)
        c = h - mean
        var = jnp.mean(c * c, axis=0, keepdims=True)
        h = c * lax.rsqrt(var + 1e-5) * g[...] + be[...]
        h = jnp.maximum(h, 0.0)
        out[...] = lax.dot_general(h, w2[...], (((1,), (1,)), ((), ())),
                                   preferred_element_type=jnp.float32) + b2[...]

    return pl.pallas_call(
        _mlp_body, out_shape=jax.ShapeDtypeStruct((n, d), jnp.float32))


def kernel(nfeat, edge_index, W1, b1, bn_gamma, bn_beta, W2, b2):
    n, d = nfeat.shape
    e = edge_index.shape[1]
    npad = _round_up(n + 1, NS * 8)
    cps = _round_up(e, NSLICE * CHUNK) // (NSLICE * CHUNK)
    epad = NSLICE * cps * CHUNK
    src = edge_index[0]
    dst = edge_index[1]
    if epad > e:
        i = jnp.arange(epad - e, dtype=jnp.int32)
        src = jnp.concatenate([src, i % n])
        dst = jnp.concatenate([dst, n + i % (npad - n)])
    src = src.reshape(NSLICE, cps, CHUNK)
    dst = dst.reshape(NSLICE, cps, CHUNK)
    zeros = jnp.zeros((npad // NS, d), jnp.float32)
    aggs = _make_sc_aggregate(n, d, npad, cps)(src, dst, nfeat, zeros)
    return _make_mlp(n, d, npad)(
        nfeat, aggs, W1, b1.reshape(1, d), bn_gamma.reshape(1, d),
        bn_beta.reshape(1, d), W2, b2.reshape(1, d))


# speedup vs baseline: 4.5860x; 1.1086x over previous
import functools

import jax
import jax.numpy as jnp
from jax import lax
from jax.experimental import pallas as pl
from jax.experimental.pallas import tpu as pltpu
from jax.experimental.pallas import tpu_sc as plsc

NC = 2
NS = 16
NW = NC * NS
CHUNK = 112
UNROLL = 6


def _round_up(x, m):
    return (x + m - 1) // m * m


@functools.lru_cache(maxsize=None)
def _make_sc_aggregate(n, d, npad, cps):
    rps = npad // NS
    assert cps % UNROLL == 0 and cps >= UNROLL

    mesh = plsc.VectorSubcoreMesh(core_axis_name="c", subcore_axis_name="s")

    scratch = (
        [pltpu.VMEM((CHUNK,), jnp.int32) for _ in range(6)]
        + [pltpu.VMEM((CHUNK,), jnp.int32) for _ in range(6)]
        + [pltpu.VMEM((CHUNK, d), jnp.float32) for _ in range(3)]
        + [pltpu.VMEM_SHARED((npad, d), jnp.float32)]
        + [pltpu.SemaphoreType.DMA for _ in range(6)]
        + [pltpu.SemaphoreType.DMA for _ in range(3)]
        + [pltpu.SemaphoreType.DMA]
    )

    @functools.partial(
        pl.kernel,
        mesh=mesh,
        out_type=jax.ShapeDtypeStruct((NC * npad, d), jnp.float32),
        scratch_types=scratch,
    )
    def sc_agg(src_hbm, dst_hbm, feat_hbm, zeros_hbm, out_hbm, *scr):
        si = scr[0:6]
        di = scr[6:12]
        rows = scr[12:15]
        agg_sh = scr[15]
        isem = scr[16:22]
        g = scr[22:25]
        sc = scr[25]

        cid = lax.axis_index("c")
        sid = lax.axis_index("s")
        wid = sid * NC + cid
        row0 = sid * rps

        pltpu.sync_copy(zeros_hbm, agg_sh.at[pl.ds(row0, rps)])
        plsc.subcore_barrier()

        def load_idx(c, lane):
            pltpu.async_copy(src_hbm.at[wid, c], si[lane], isem[lane])
            pltpu.async_copy(dst_hbm.at[wid, c], di[lane], isem[lane])

        def wait_idx(lane):
            pltpu.make_async_copy(src_hbm.at[wid, 0], si[lane], isem[lane]).wait()
            pltpu.make_async_copy(dst_hbm.at[wid, 0], di[lane], isem[lane]).wait()

        def gather(lane_i, lane_r):
            pltpu.async_copy(feat_hbm.at[si[lane_i]], rows[lane_r], g[lane_r])

        def wait_gather(lane_r):
            pltpu.make_async_copy(
                feat_hbm.at[si[0]], rows[lane_r], g[lane_r]).wait()

        def drain_scatter():
            pltpu.make_async_copy(rows[0], agg_sh.at[di[0]], sc).wait()

        for q in range(5):
            load_idx(q, q)
        wait_idx(0)
        gather(0, 0)
        wait_idx(1)
        gather(1, 1)

        def body(k, carry):
            for q in range(UNROLL):
                s = k * UNROLL + q

                @pl.when(s >= 1)
                def _():
                    drain_scatter()

                @pl.when(s + 2 < cps)
                def _():
                    wait_idx((q + 2) % 6)
                    gather((q + 2) % 6, (q + 2) % 3)

                @pl.when(s + 5 < cps)
                def _():
                    load_idx(s + 5, (q + 5) % 6)

                wait_gather(q % 3)
                pltpu.async_copy(rows[q % 3], agg_sh.at[di[q]], sc, add=True)
            return carry
        lax.fori_loop(0, cps // UNROLL, body, 0)

        drain_scatter()

        plsc.subcore_barrier()

        pltpu.sync_copy(agg_sh.at[pl.ds(row0, rps)],
                        out_hbm.at[pl.ds(cid * npad + row0, rps)])

    return sc_agg


def _make_mlp(n, d, npad):
    def _mlp_body(feat, aggs, w1, b1, g, be, w2, b2, out):
        h = feat[...] + aggs[:n, :] + aggs[npad:npad + n, :]
        h = lax.dot_general(h, w1[...], (((1,), (1,)), ((), ())),
                            preferred_element_type=jnp.float32) + b1[...]
        mean = jnp.mean(h, axis=0, keepdims=True)
        c = h - mean
        var = jnp.mean(c * c, axis=0, keepdims=True)
        h = c * lax.rsqrt(var + 1e-5) * g[...] + be[...]
        h = jnp.maximum(h, 0.0)
        out[...] = lax.dot_general(h, w2[...], (((1,), (1,)), ((), ())),
                                   preferred_element_type=jnp.float32) + b2[...]

    return pl.pallas_call(
        _mlp_body, out_shape=jax.ShapeDtypeStruct((n, d), jnp.float32))


def kernel(nfeat, edge_index, W1, b1, bn_gamma, bn_beta, W2, b2):
    n, d = nfeat.shape
    e = edge_index.shape[1]
    npad = _round_up(n + 1, NS * 8)
    cps = _round_up(-(-e // (NW * CHUNK)), UNROLL)
    epad = NW * cps * CHUNK
    src = edge_index[0]
    dst = edge_index[1]
    if epad > e:
        i = jnp.arange(epad - e, dtype=jnp.int32)
        src = jnp.concatenate([src, i % n])
        dst = jnp.concatenate([dst, n + i % (npad - n)])
    src = src.reshape(NW, cps, CHUNK)
    dst = dst.reshape(NW, cps, CHUNK)
    zeros = jnp.zeros((npad // NS, d), jnp.float32)
    aggs = _make_sc_aggregate(n, d, npad, cps)(src, dst, nfeat, zeros)
    return _make_mlp(n, d, npad)(
        nfeat, aggs, W1, b1.reshape(1, d), bn_gamma.reshape(1, d),
        bn_beta.reshape(1, d), W2, b2.reshape(1, d))


# speedup vs baseline: 5.2301x; 1.1404x over previous
import functools

import jax
import jax.numpy as jnp
from jax import lax
from jax.experimental import pallas as pl
from jax.experimental.pallas import tpu as pltpu
from jax.experimental.pallas import tpu_sc as plsc

NC = 2
NS = 16
NW = NC * NS
CHUNK = 128
UNROLL = 6


def _round_up(x, m):
    return (x + m - 1) // m * m


@functools.lru_cache(maxsize=None)
def _make_sc_aggregate(n, d, npad, total_chunks, cps):
    rps = npad // NS
    assert cps % UNROLL == 0 and cps >= UNROLL

    mesh = plsc.VectorSubcoreMesh(core_axis_name="c", subcore_axis_name="s")

    scratch = (
        [pltpu.VMEM((CHUNK,), jnp.int32) for _ in range(3)]
        + [pltpu.VMEM((CHUNK,), jnp.int32) for _ in range(3)]
        + [pltpu.VMEM((CHUNK, d), jnp.float32) for _ in range(3)]
        + [pltpu.VMEM_SHARED((npad, d), jnp.float32)]
        + [pltpu.SemaphoreType.DMA for _ in range(3)]
        + [pltpu.SemaphoreType.DMA for _ in range(3)]
        + [pltpu.SemaphoreType.DMA for _ in range(3)]
        + [pltpu.SemaphoreType.DMA]
    )

    @functools.partial(
        pl.kernel,
        mesh=mesh,
        out_type=jax.ShapeDtypeStruct((NC * npad, d), jnp.float32),
        scratch_types=scratch,
    )
    def sc_agg(edge_hbm, feat_hbm, zeros_hbm, out_hbm, *scr):
        si = scr[0:3]
        di = scr[3:6]
        rows = scr[6:9]
        agg_sh = scr[9]
        ssem = scr[10:13]
        dsem = scr[13:16]
        g = scr[16:19]
        sc = scr[19]

        cid = lax.axis_index("c")
        sid = lax.axis_index("s")
        wid = sid * NC + cid
        row0 = sid * rps
        vlim = (total_chunks + NW - 1 - wid) // NW

        def load_si(s_, lane):
            pltpu.async_copy(edge_hbm.at[s_ * NW + wid, 0], si[lane], ssem[lane])

        def load_di(s_, lane):
            pltpu.async_copy(edge_hbm.at[s_ * NW + wid, 1], di[lane], dsem[lane])

        def wait_si(lane):
            pltpu.make_async_copy(edge_hbm.at[0, 0], si[lane], ssem[lane]).wait()

        def wait_di(lane):
            pltpu.make_async_copy(edge_hbm.at[0, 1], di[lane], dsem[lane]).wait()

        def gather(lane_i, lane_r):
            pltpu.async_copy(feat_hbm.at[si[lane_i]], rows[lane_r], g[lane_r])

        def wait_gather(lane_r):
            pltpu.make_async_copy(
                feat_hbm.at[si[0]], rows[lane_r], g[lane_r]).wait()

        def drain_scatter():
            pltpu.make_async_copy(rows[0], agg_sh.at[di[0]], sc).wait()

        for q in range(3):
            load_si(q, q)
        load_di(0, 0)
        load_di(1, 1)
        wait_si(0)
        gather(0, 0)
        wait_si(1)
        gather(1, 1)

        pltpu.sync_copy(zeros_hbm, agg_sh.at[pl.ds(row0, rps)])
        plsc.subcore_barrier()

        def body(k, carry):
            for q in range(UNROLL):
                s = k * UNROLL + q

                @pl.when(jnp.logical_and(s >= 1, s <= vlim))
                def _():
                    drain_scatter()

                @pl.when(s + 2 < vlim)
                def _():
                    wait_si((q + 2) % 3)
                    gather((q + 2) % 3, (q + 2) % 3)
                    load_di(s + 2, (q + 2) % 3)

                @pl.when(s < vlim)
                def _():
                    wait_gather(q % 3)

                @pl.when(s + 3 < vlim)
                def _():
                    load_si(s + 3, q % 3)

                @pl.when(s < vlim)
                def _():
                    wait_di(q % 3)
                    pltpu.async_copy(rows[q % 3], agg_sh.at[di[q % 3]],
                                     sc, add=True)
            return carry
        lax.fori_loop(0, cps // UNROLL, body, 0)

        plsc.subcore_barrier()

        pltpu.sync_copy(agg_sh.at[pl.ds(row0, rps)],
                        out_hbm.at[pl.ds(cid * npad + row0, rps)])

    return sc_agg


def _make_mlp(n, d, npad):
    def _mlp_body(feat, aggs, w1, b1, g, be, w2, b2, out):
        h = feat[...] + aggs[:n, :] + aggs[npad:npad + n, :]
        h = lax.dot_general(h, w1[...], (((1,), (1,)), ((), ())),
                            preferred_element_type=jnp.float32) + b1[...]
        mean = jnp.mean(h, axis=0, keepdims=True)
        c = h - mean
        var = jnp.mean(c * c, axis=0, keepdims=True)
        h = c * lax.rsqrt(var + 1e-5) * g[...] + be[...]
        h = jnp.maximum(h, 0.0)
        out[...] = lax.dot_general(h, w2[...], (((1,), (1,)), ((), ())),
                                   preferred_element_type=jnp.float32) + b2[...]

    return pl.pallas_call(
        _mlp_body, out_shape=jax.ShapeDtypeStruct((n, d), jnp.float32))


def kernel(nfeat, edge_index, W1, b1, bn_gamma, bn_beta, W2, b2):
    n, d = nfeat.shape
    e = edge_index.shape[1]
    npad = _round_up(n + 1, NS * 8)
    ec = _round_up(e, CHUNK)
    if ec > e:
        i = jnp.arange(ec - e, dtype=jnp.int32)
        pad = jnp.stack([i % n, n + i % (npad - n)])
        edge_index = jnp.concatenate([edge_index, pad], axis=1)
    total_chunks = ec // CHUNK
    cps = _round_up(-(-total_chunks // NW), UNROLL)
    edge_t = edge_index.reshape(2, total_chunks, CHUNK).transpose(1, 0, 2)
    zeros = jnp.zeros((npad // NS, d), jnp.float32)
    aggs = _make_sc_aggregate(n, d, npad, total_chunks, cps)(
        edge_t, nfeat, zeros)
    return _make_mlp(n, d, npad)(
        nfeat, aggs, W1, b1.reshape(1, d), bn_gamma.reshape(1, d),
        bn_beta.reshape(1, d), W2, b2.reshape(1, d))
